# baseline (speedup 1.0000x reference)
import functools

import jax
import jax.numpy as jnp
from jax import lax
from jax.experimental import pallas as pl
from jax.experimental.pallas import tpu as pltpu


def _se_kernel(x_ref, w1_ref, b1_ref, w2t_ref, b2_ref, out_ref, acc_ref, *,
               inv_hw):
    k = pl.program_id(1)
    n_k = pl.num_programs(1)

    @pl.when(k == 0)
    def _():
        acc_ref[...] = jnp.zeros_like(acc_ref)

    xv = x_ref[...]
    acc_ref[...] += jnp.sum(xv.astype(jnp.float32), axis=(0, 1))

    @pl.when(k == n_k - 1)
    def _():
        pooled = acc_ref[...] * inv_hw
        h = lax.dot_general(pooled, w1_ref[...], (((1,), (1,)), ((), ())),
                            preferred_element_type=jnp.float32)
        h = jnp.maximum(h + b1_ref[...], 0.0)
        y = lax.dot_general(h, w2t_ref[...], (((1,), (0,)), ((), ())),
                            preferred_element_type=jnp.float32)
        out_ref[...] = jax.nn.sigmoid(y + b2_ref[...])


def kernel(x, w1, b1, w2, b2):
    N, C, H, W = x.shape
    hidden = w1.shape[0]
    channel = w2.shape[0]
    itemsize = jnp.dtype(x.dtype).itemsize

    xt = jnp.transpose(x, (2, 3, 0, 1))

    tn = N
    for d in range(1, N + 1):
        if N % d == 0 and d % 8 == 0 and N // d >= 2:
            tn = d
    if tn == N and N > 1:
        for d in range(1, N + 1):
            if N % d == 0 and N // d >= 2:
                tn = d
    n_par = N // tn

    target = 12 * 1024 * 1024
    row_bytes = W * tn * C * itemsize
    th = H
    best = None
    for d in range(1, H + 1):
        if H % d == 0:
            score = abs(d * row_bytes - target)
            if best is None or score < best:
                best, th = score, d
    n_k = H // th

    b1_r = b1.reshape(1, hidden)
    b2_r = b2.reshape(1, channel)
    w2_t = w2.T

    kernel_fn = functools.partial(_se_kernel, inv_hw=1.0 / float(H * W))

    x_block_bytes = th * W * tn * C * itemsize
    w_bytes = 4 * (C * hidden + hidden + hidden * channel + channel)
    vmem_limit = int(min(60 * 1024 * 1024,
                         2 * x_block_bytes + 2 * w_bytes
                         + 8 * tn * channel + (4 << 20)))

    cost = pl.CostEstimate(
        flops=int(N * C * H * W + 2 * N * C * hidden
                  + 2 * N * hidden * channel),
        transcendentals=int(N * channel),
        bytes_accessed=int(N * C * H * W * itemsize + n_par * w_bytes
                           + 4 * N * channel),
    )

    out = pl.pallas_call(
        kernel_fn,
        out_shape=jax.ShapeDtypeStruct((N, channel), jnp.float32),
        grid=(n_par, n_k),
        in_specs=[
            pl.BlockSpec((th, W, tn, C), lambda n, k: (k, 0, n, 0)),
            pl.BlockSpec((hidden, C), lambda n, k: (0, 0)),
            pl.BlockSpec((1, hidden), lambda n, k: (0, 0)),
            pl.BlockSpec((hidden, channel), lambda n, k: (0, 0)),
            pl.BlockSpec((1, channel), lambda n, k: (0, 0)),
        ],
        out_specs=pl.BlockSpec((tn, channel), lambda n, k: (n, 0)),
        scratch_shapes=[pltpu.VMEM((tn, C), jnp.float32)],
        compiler_params=pltpu.CompilerParams(
            dimension_semantics=("parallel", "arbitrary"),
            vmem_limit_bytes=vmem_limit,
        ),
        cost_estimate=cost,
    )(xt, w1, b1_r, w2_t, b2_r)

    return out.reshape(-1, channel, 1, 1)


# speedup vs baseline: 1.0187x; 1.0187x over previous
import functools

import jax
import jax.numpy as jnp
from jax import lax
from jax.experimental import pallas as pl
from jax.experimental.pallas import tpu as pltpu


def _se_kernel(x_ref, w1_ref, b1_ref, w2t_ref, b2_ref, out_ref, acc_ref, *,
               inv_hw):
    k = pl.program_id(1)
    n_k = pl.num_programs(1)

    @pl.when(k == 0)
    def _():
        acc_ref[...] = jnp.zeros_like(acc_ref)

    xv = x_ref[...]
    acc_ref[...] += jnp.sum(xv.astype(jnp.float32), axis=(0, 1))

    @pl.when(k == n_k - 1)
    def _():
        pooled = acc_ref[...] * inv_hw
        h = lax.dot_general(pooled, w1_ref[...], (((1,), (1,)), ((), ())),
                            preferred_element_type=jnp.float32)
        h = jnp.maximum(h + b1_ref[...], 0.0)
        y = lax.dot_general(h, w2t_ref[...], (((1,), (0,)), ((), ())),
                            preferred_element_type=jnp.float32)
        out_ref[...] = jax.nn.sigmoid(y + b2_ref[...])


def kernel(x, w1, b1, w2, b2):
    N, C, H, W = x.shape
    hidden = w1.shape[0]
    channel = w2.shape[0]
    itemsize = jnp.dtype(x.dtype).itemsize

    xt = jnp.transpose(x, (2, 3, 0, 1))

    tn = N
    for d in range(1, N + 1):
        if N % d == 0 and d % 8 == 0 and N // d >= 2:
            tn = d
    if tn == N and N > 1:
        for d in range(1, N + 1):
            if N % d == 0 and N // d >= 2:
                tn = d
    n_par = N // tn

    target = 6 * 1024 * 1024
    row_bytes = W * tn * C * itemsize
    th = H
    best = None
    for d in range(1, H + 1):
        if H % d == 0:
            score = abs(d * row_bytes - target)
            if best is None or score < best:
                best, th = score, d
    n_k = H // th

    b1_r = b1.reshape(1, hidden)
    b2_r = b2.reshape(1, channel)
    w2_t = w2.T

    kernel_fn = functools.partial(_se_kernel, inv_hw=1.0 / float(H * W))

    x_block_bytes = th * W * tn * C * itemsize
    w_bytes = 4 * (C * hidden + hidden + hidden * channel + channel)
    vmem_limit = int(min(60 * 1024 * 1024,
                         2 * x_block_bytes + 2 * w_bytes
                         + 8 * tn * channel + (4 << 20)))

    cost = pl.CostEstimate(
        flops=int(N * C * H * W + 2 * N * C * hidden
                  + 2 * N * hidden * channel),
        transcendentals=int(N * channel),
        bytes_accessed=int(N * C * H * W * itemsize + n_par * w_bytes
                           + 4 * N * channel),
    )

    out = pl.pallas_call(
        kernel_fn,
        out_shape=jax.ShapeDtypeStruct((N, channel), jnp.float32),
        grid=(n_par, n_k),
        in_specs=[
            pl.BlockSpec((th, W, tn, C), lambda n, k: (k, 0, n, 0)),
            pl.BlockSpec((hidden, C), lambda n, k: (0, 0)),
            pl.BlockSpec((1, hidden), lambda n, k: (0, 0)),
            pl.BlockSpec((hidden, channel), lambda n, k: (0, 0)),
            pl.BlockSpec((1, channel), lambda n, k: (0, 0)),
        ],
        out_specs=pl.BlockSpec((tn, channel), lambda n, k: (n, 0)),
        scratch_shapes=[pltpu.VMEM((tn, C), jnp.float32)],
        compiler_params=pltpu.CompilerParams(
            dimension_semantics=("parallel", "arbitrary"),
            vmem_limit_bytes=vmem_limit,
        ),
        cost_estimate=cost,
    )(xt, w1, b1_r, w2_t, b2_r)

    return out.reshape(-1, channel, 1, 1)


# speedup vs baseline: 1.0900x; 1.0699x over previous
import functools

import jax
import jax.numpy as jnp
from jax import lax
from jax.experimental import pallas as pl
from jax.experimental.pallas import tpu as pltpu


def _se_kernel(x_ref, p_ref, out_ref, acc_ref, *, inv_hw, hidden, c_in):
    k = pl.program_id(1)
    n_k = pl.num_programs(1)

    @pl.when(k == 0)
    def _():
        acc_ref[...] = jnp.zeros_like(acc_ref)

    xv = x_ref[...]
    acc_ref[...] += jnp.sum(xv.astype(jnp.float32), axis=(0, 1))

    @pl.when(k == n_k - 1)
    def _():
        channel = out_ref.shape[-1]
        pooled = acc_ref[...] * inv_hw
        w1v = p_ref[0:hidden, 0:c_in]
        w2tv = p_ref[hidden:2 * hidden, 0:channel]
        b1v = p_ref[2 * hidden:2 * hidden + 1, 0:hidden]
        b2v = p_ref[2 * hidden + 1:2 * hidden + 2, 0:channel]
        h = lax.dot_general(pooled, w1v, (((1,), (1,)), ((), ())),
                            preferred_element_type=jnp.float32)
        h = jnp.maximum(h + b1v, 0.0)
        y = lax.dot_general(h, w2tv, (((1,), (0,)), ((), ())),
                            preferred_element_type=jnp.float32)
        out_ref[...] = jax.nn.sigmoid(y + b2v)


def kernel(x, w1, b1, w2, b2):
    N, C, H, W = x.shape
    hidden = w1.shape[0]
    channel = w2.shape[0]
    itemsize = jnp.dtype(x.dtype).itemsize

    xt = jnp.transpose(x, (2, 3, 0, 1))

    tn = N
    for d in range(1, N + 1):
        if N % d == 0 and d % 8 == 0 and N // d >= 2:
            tn = d
    if tn == N and N > 1:
        for d in range(1, N + 1):
            if N % d == 0 and N // d >= 2:
                tn = d
    n_par = N // tn

    target = 6 * 1024 * 1024
    row_bytes = W * tn * C * itemsize
    th = H
    best = None
    for d in range(1, H + 1):
        if H % d == 0:
            score = abs(d * row_bytes - target)
            if best is None or score < best:
                best, th = score, d
    n_k = H // th

    cmax = max(C, channel)
    w1_p = jnp.pad(w1, ((0, 0), (0, cmax - C))) if C < cmax else w1
    w2_t = w2.T
    w2_p = (jnp.pad(w2_t, ((0, 0), (0, cmax - channel)))
            if channel < cmax else w2_t)
    b1_p = jnp.pad(b1, (0, cmax - hidden))[None]
    b2_p = (jnp.pad(b2, (0, cmax - channel)) if channel < cmax else b2)[None]
    packed = jnp.concatenate([w1_p, w2_p, b1_p, b2_p], axis=0)
    p_rows = 2 * hidden + 2

    kernel_fn = functools.partial(_se_kernel, inv_hw=1.0 / float(H * W),
                                  hidden=hidden, c_in=C)

    x_block_bytes = th * W * tn * C * itemsize
    w_bytes = 4 * (C * hidden + hidden + hidden * channel + channel)
    vmem_limit = int(min(60 * 1024 * 1024,
                         2 * x_block_bytes + 2 * w_bytes
                         + 8 * tn * channel + (4 << 20)))

    cost = pl.CostEstimate(
        flops=int(N * C * H * W + 2 * N * C * hidden
                  + 2 * N * hidden * channel),
        transcendentals=int(N * channel),
        bytes_accessed=int(N * C * H * W * itemsize + n_par * w_bytes
                           + 4 * N * channel),
    )

    out = pl.pallas_call(
        kernel_fn,
        out_shape=jax.ShapeDtypeStruct((N, channel), jnp.float32),
        grid=(n_par, n_k),
        in_specs=[
            pl.BlockSpec((th, W, tn, C), lambda n, k: (k, 0, n, 0)),
            pl.BlockSpec((p_rows, cmax), lambda n, k: (0, 0)),
        ],
        out_specs=pl.BlockSpec((tn, channel), lambda n, k: (n, 0)),
        scratch_shapes=[pltpu.VMEM((tn, C), jnp.float32)],
        compiler_params=pltpu.CompilerParams(
            dimension_semantics=("parallel", "arbitrary"),
            vmem_limit_bytes=vmem_limit,
        ),
        cost_estimate=cost,
    )(xt, packed)

    return out.reshape(-1, channel, 1, 1)
